# baseline (speedup 1.0000x reference)
import jax
import jax.numpy as jnp
from jax.experimental import pallas as pl
from jax.experimental.pallas import tpu as pltpu

_EPS = 1e-8


def _tile(n, target):
    if n <= target:
        return n
    t = (target // 128) * 128
    while t >= 128:
        if n % t == 0:
            return t
        t -= 128
    return n


def _fused_kernel(xyz1_ref, xyz2_ref, pts1_ref, pts2_ref,
                  w1_ref, w2_ref, b12_ref, o_ref):
    x1 = xyz1_ref[0]
    x2 = xyz2_ref[0]

    p1 = jnp.dot(w1_ref[...].astype(jnp.bfloat16),
                 pts1_ref[0].astype(jnp.bfloat16),
                 preferred_element_type=jnp.float32)
    p2 = jnp.dot(w2_ref[...].astype(jnp.bfloat16),
                 pts2_ref[0].astype(jnp.bfloat16),
                 preferred_element_type=jnp.float32)

    d = None
    for c in range(3):
        col = jnp.transpose(x1[c:c + 1, :])
        diff = col - x2[c:c + 1, :]
        sq = diff * diff
        d = sq if d is None else d + sq

    M = d.shape[0]
    inf = jnp.float32(jnp.inf)
    a = d[0:8, :]
    b = jnp.full_like(a, inf)
    c3 = jnp.full_like(a, inf)
    for i in range(8, M, 8):
        v = d[i:i + 8, :]
        a, t = jnp.minimum(a, v), jnp.maximum(a, v)
        b, t = jnp.minimum(b, t), jnp.maximum(b, t)
        c3 = jnp.minimum(c3, t)

    def _merge3(a1, b1, c1, a2, b2, c2):
        lo = jnp.minimum(a1, a2)
        t = jnp.maximum(a1, a2)
        u = jnp.minimum(b1, b2)
        mid = jnp.minimum(t, u)
        hi = jnp.minimum(jnp.minimum(c1, c2), jnp.maximum(t, u))
        return lo, mid, hi

    h = 4
    while h >= 1:
        a, b, c3 = _merge3(a[:h], b[:h], c3[:h],
                           a[h:2 * h], b[h:2 * h], c3[h:2 * h])
        h //= 2
    m1, m2, m3 = a, b, c3

    wmat = jnp.where(d <= m3, 1.0 / (d + _EPS), 0.0)
    inv_norm = 1.0 / (1.0 / (m1 + _EPS) + 1.0 / (m2 + _EPS)
                      + 1.0 / (m3 + _EPS))

    interp = jnp.dot(p1.astype(jnp.bfloat16), wmat.astype(jnp.bfloat16),
                     preferred_element_type=jnp.float32)
    bc = jnp.transpose(b12_ref[...])
    o_ref[...] = (interp * inv_norm + p2 + bc)[None]


def kernel(xyz_1, xyz_2, points_1, points_2, w1, b1, w2, b2):
    B, _, M = xyz_1.shape
    N = xyz_2.shape[2]
    Cout, Cin = w1.shape
    TN = _tile(N, 2048)
    b12 = (b1 + b2).reshape(1, Cout)
    new_points = pl.pallas_call(
        _fused_kernel,
        out_shape=jax.ShapeDtypeStruct((B, Cout, N), points_2.dtype),
        grid_spec=pltpu.PrefetchScalarGridSpec(
            num_scalar_prefetch=0,
            grid=(B, N // TN),
            in_specs=[
                pl.BlockSpec((1, 3, M), lambda b, n: (b, 0, 0)),
                pl.BlockSpec((1, 3, TN), lambda b, n: (b, 0, n)),
                pl.BlockSpec((1, Cin, M), lambda b, n: (b, 0, 0)),
                pl.BlockSpec((1, Cout, TN), lambda b, n: (b, 0, n)),
                pl.BlockSpec((Cout, Cin), lambda b, n: (0, 0)),
                pl.BlockSpec((Cout, Cout), lambda b, n: (0, 0)),
                pl.BlockSpec((1, Cout), lambda b, n: (0, 0)),
            ],
            out_specs=pl.BlockSpec((1, Cout, TN), lambda b, n: (b, 0, n)),
        ),
        compiler_params=pltpu.CompilerParams(
            dimension_semantics=("parallel", "parallel")),
    )(xyz_1, xyz_2, points_1, points_2, w1, w2, b12)
    return xyz_2, new_points
